# Initial kernel scaffold; baseline (speedup 1.0000x reference)
import jax
import jax.numpy as jnp
from jax.experimental import pallas as pl


def kernel(x, embedding_weight):
    raise NotImplementedError("write your pallas kernel here")


# speedup vs baseline: 1.0244x; 1.0244x over previous
import jax
import jax.numpy as jnp
from jax.experimental import pallas as pl


def _body(e_ref, x_ref, o_ref):
    i = pl.program_id(0)
    n_rows = x_ref.shape[1]
    max_len, d = e_ref.shape
    half = max_len // 2
    g = jax.lax.broadcasted_iota(jnp.int32, (n_rows, max_len), 0)
    k = jax.lax.broadcasted_iota(jnp.int32, (n_rows, max_len), 1)
    idx = jnp.clip(g - i + half, 0, max_len - 1)
    onehot = (k == idx).astype(jnp.float32)
    s = jnp.dot(onehot, e_ref[...], preferred_element_type=jnp.float32)
    o_ref[...] = x_ref[...] + s[None, :, None, :]


def kernel(x, embedding_weight):
    t = x.shape[0]
    ncv = x.shape[1] // t
    d = x.shape[2]
    max_len = embedding_weight.shape[0]
    x_r = x.reshape(t, t, ncv, d)

    out = pl.pallas_call(
        _body,
        grid=(t,),
        in_specs=[
            pl.BlockSpec((max_len, d), lambda i: (0, 0)),
            pl.BlockSpec((1, t, ncv, d), lambda i: (i, 0, 0, 0)),
        ],
        out_specs=pl.BlockSpec((1, t, ncv, d), lambda i: (i, 0, 0, 0)),
        out_shape=jax.ShapeDtypeStruct((t, t, ncv, d), jnp.float32),
    )(embedding_weight, x_r)
    return out.reshape(x.shape)


# speedup vs baseline: 3.8786x; 3.7864x over previous
import jax
import jax.numpy as jnp
from jax.experimental import pallas as pl
from jax.experimental.pallas import tpu as pltpu


def _body(e_ref, x_ref, o_ref, rep_ref):
    i = pl.program_id(0)
    max_len, d = e_ref.shape
    n = rep_ref.shape[1]
    half = max_len // 2

    @pl.when(i == 0)
    def _build_rep():
        j = jax.lax.broadcasted_iota(jnp.int32, rep_ref.shape, 0)
        g = jax.lax.broadcasted_iota(jnp.int32, rep_ref.shape, 1)
        ncv = rep_ref.shape[0] // n
        rep_ref[...] = (j // ncv == g).astype(jnp.bfloat16)

    r = jax.lax.broadcasted_iota(jnp.int32, (n, max_len), 0)
    k = jax.lax.broadcasted_iota(jnp.int32, (n, max_len), 1)
    idx = jnp.clip(r - i + half, 0, max_len - 1)
    onehot = (k == idx).astype(jnp.float32)
    s = jnp.dot(onehot, e_ref[...], preferred_element_type=jnp.float32)
    addend = jnp.dot(rep_ref[...], s.astype(jnp.bfloat16),
                     preferred_element_type=jnp.float32)
    o_ref[...] = x_ref[...] + addend[None, :, :]


def kernel(x, embedding_weight):
    t = x.shape[0]
    jn = x.shape[1]
    d = x.shape[2]
    max_len = embedding_weight.shape[0]

    return pl.pallas_call(
        _body,
        grid=(t,),
        in_specs=[
            pl.BlockSpec((max_len, d), lambda i: (0, 0)),
            pl.BlockSpec((1, jn, d), lambda i: (i, 0, 0)),
        ],
        out_specs=pl.BlockSpec((1, jn, d), lambda i: (i, 0, 0)),
        out_shape=jax.ShapeDtypeStruct(x.shape, x.dtype),
        scratch_shapes=[pltpu.VMEM((jn, t), jnp.bfloat16)],
    )(embedding_weight, x)


# speedup vs baseline: 6.6089x; 1.7039x over previous
import jax
import jax.numpy as jnp
from jax.experimental import pallas as pl
from jax.experimental.pallas import tpu as pltpu


_BI = 4


def _body(e_ref, x_ref, o_ref, rep_ref):
    i0 = pl.program_id(0) * _BI
    max_len, d = e_ref.shape
    n = rep_ref.shape[1]
    half = max_len // 2

    @pl.when(i0 == 0)
    def _build_rep():
        j = jax.lax.broadcasted_iota(jnp.int32, rep_ref.shape, 0)
        g = jax.lax.broadcasted_iota(jnp.int32, rep_ref.shape, 1)
        ncv = rep_ref.shape[0] // n
        rep_ref[...] = (j // ncv == g).astype(jnp.bfloat16)

    r = jax.lax.broadcasted_iota(jnp.int32, (n, max_len), 0)
    k = jax.lax.broadcasted_iota(jnp.int32, (n, max_len), 1)
    for bi in range(_BI):
        idx = jnp.clip(r - (i0 + bi) + half, 0, max_len - 1)
        onehot = (k == idx).astype(jnp.float32)
        s = jnp.dot(onehot, e_ref[...], preferred_element_type=jnp.float32)
        addend = jnp.dot(rep_ref[...], s.astype(jnp.bfloat16),
                         preferred_element_type=jnp.float32)
        o_ref[bi, :, :] = x_ref[bi, :, :] + addend


def kernel(x, embedding_weight):
    t = x.shape[0]
    jn = x.shape[1]
    d = x.shape[2]
    max_len = embedding_weight.shape[0]

    return pl.pallas_call(
        _body,
        grid=(t // _BI,),
        in_specs=[
            pl.BlockSpec((max_len, d), lambda i: (0, 0)),
            pl.BlockSpec((_BI, jn, d), lambda i: (i, 0, 0)),
        ],
        out_specs=pl.BlockSpec((_BI, jn, d), lambda i: (i, 0, 0)),
        out_shape=jax.ShapeDtypeStruct(x.shape, x.dtype),
        scratch_shapes=[pltpu.VMEM((jn, t), jnp.bfloat16)],
    )(embedding_weight, x)


# speedup vs baseline: 7.0518x; 1.0670x over previous
import jax
import jax.numpy as jnp
from jax.experimental import pallas as pl
from jax.experimental.pallas import tpu as pltpu


_BI = 8


def _body(e_ref, x_ref, o_ref, rep_ref):
    i0 = pl.program_id(0) * _BI
    max_len, d = e_ref.shape
    n = rep_ref.shape[1]
    half = max_len // 2

    @pl.when(i0 == 0)
    def _build_rep():
        j = jax.lax.broadcasted_iota(jnp.int32, rep_ref.shape, 0)
        g = jax.lax.broadcasted_iota(jnp.int32, rep_ref.shape, 1)
        ncv = rep_ref.shape[0] // n
        rep_ref[...] = (j // ncv == g).astype(jnp.bfloat16)

    r = jax.lax.broadcasted_iota(jnp.int32, (n, max_len), 0)
    k = jax.lax.broadcasted_iota(jnp.int32, (n, max_len), 1)
    for bi in range(_BI):
        idx = jnp.clip(r - (i0 + bi) + half, 0, max_len - 1)
        onehot = (k == idx).astype(jnp.float32)
        s = jnp.dot(onehot, e_ref[...], preferred_element_type=jnp.float32)
        addend = jnp.dot(rep_ref[...], s.astype(jnp.bfloat16),
                         preferred_element_type=jnp.float32)
        o_ref[bi, :, :] = x_ref[bi, :, :] + addend


def kernel(x, embedding_weight):
    t = x.shape[0]
    jn = x.shape[1]
    d = x.shape[2]
    max_len = embedding_weight.shape[0]

    return pl.pallas_call(
        _body,
        grid=(t // _BI,),
        in_specs=[
            pl.BlockSpec((max_len, d), lambda i: (0, 0)),
            pl.BlockSpec((_BI, jn, d), lambda i: (i, 0, 0)),
        ],
        out_specs=pl.BlockSpec((_BI, jn, d), lambda i: (i, 0, 0)),
        out_shape=jax.ShapeDtypeStruct(x.shape, x.dtype),
        scratch_shapes=[pltpu.VMEM((jn, t), jnp.bfloat16)],
    )(embedding_weight, x)
